# baseline (speedup 1.0000x reference)
import functools

import jax
import jax.numpy as jnp
from jax import lax
from jax.experimental import pallas as pl
from jax.experimental.pallas import tpu as pltpu
from jax.experimental.pallas import tpu_sc as plsc

B = 4096
L = 200
D = 64
V = 1000000
NC = 2
NS = 16
NW = NC * NS
BPW = B // NW
NCH = D // 16
NG = L // 16
TAIL = L - 16 * NG
UN = 4

NBLK = V // 128
VPAD = V - NBLK * 128

_mesh = plsc.VectorSubcoreMesh(core_axis_name="c", subcore_axis_name="s")


@functools.partial(
    pl.kernel,
    mesh=_mesh,
    compiler_params=pltpu.CompilerParams(needs_layout_passes=False),
    out_type=jax.ShapeDtypeStruct((V, D), jnp.float32),
    scratch_types=[
        pltpu.VMEM((D, 128), jnp.float32),
        pltpu.VMEM((D, 128), jnp.float32),
        pltpu.VMEM((128, D), jnp.float32),
        pltpu.VMEM((128, D), jnp.float32),
        pltpu.SemaphoreType.DMA,
        pltpu.SemaphoreType.DMA,
        pltpu.SemaphoreType.DMA,
        pltpu.SemaphoreType.DMA,
    ],
)
def _repack(tt_hbm, tail_hbm, out_hbm, in0, in1, tr0, tr1, si0, si1, so0, so1):
    wid = lax.axis_index("s") * NC + lax.axis_index("c")
    rows_c = [lax.iota(jnp.int32, 16) + 16 * c for c in range(NCH)]

    def in_desc(bid, buf, sem):
        return pltpu.make_async_copy(
            tt_hbm.at[pl.ds(0, D), pl.ds(bid * 128, 128)], buf, sem
        )

    def out_desc(bid, buf, sem):
        return pltpu.make_async_copy(
            buf, out_hbm.at[pl.ds(bid * 128, 128)], sem
        )

    def transpose(src, dst):
        def body(m, carry):
            for half in range(2):
                cols = jnp.full((16,), 2 * m + half, jnp.int32)
                for c in range(NCH):
                    v = plsc.load_gather(src, [rows_c[c], cols])
                    dst[2 * m + half, pl.ds(16 * c, 16)] = v
            return carry

        lax.fori_loop(0, 64, body, 0)

    in_desc(wid, in0, si0).start()
    in_desc(NW + wid, in1, si1).start()
    in_desc(wid, in0, si0).wait()
    transpose(in0, tr0)
    out_desc(wid, tr0, so0).start()
    in_desc(NW + wid, in1, si1).wait()
    transpose(in1, tr1)
    out_desc(NW + wid, tr1, so1).start()
    in_desc(2 * NW + wid, in0, si0).start()

    def outer(i, carry):
        b0 = NW * (2 * i) + wid
        b1 = NW * (2 * i + 1) + wid
        b2 = jnp.minimum(NW * (2 * i + 2) + wid, NBLK - 1)
        in_desc(b1, in1, si1).start()
        in_desc(b0, in0, si0).wait()
        out_desc(b0, tr0, so0).wait()
        transpose(in0, tr0)
        out_desc(b0, tr0, so0).start()
        in_desc(b2, in0, si0).start()
        in_desc(b1, in1, si1).wait()
        out_desc(b1, tr1, so1).wait()
        transpose(in1, tr1)
        out_desc(b1, tr1, so1).start()
        return carry

    lax.fori_loop(1, 122, outer, 0)
    in_desc(jnp.int32(NBLK - 1), in0, si0).wait()

    bid_t = NW * 244 + wid

    @pl.when(bid_t < NBLK)
    def _():
        in_desc(bid_t, in0, si0).start()
        in_desc(bid_t, in0, si0).wait()
        out_desc(bid_t, tr0, so0).wait()
        transpose(in0, tr0)
        out_desc(bid_t, tr0, so0).start()
        out_desc(bid_t, tr0, so0).wait()
        out_desc(jnp.int32(0), tr1, so1).wait()

    @pl.when(bid_t >= NBLK)
    def _():
        out_desc(jnp.int32(0), tr0, so0).wait()
        out_desc(jnp.int32(0), tr1, so1).wait()

    @pl.when(wid == NW - 1)
    def _():
        pltpu.sync_copy(tail_hbm, tr0.at[pl.ds(0, V - 128 * NBLK)])
        pltpu.sync_copy(
            tr0.at[pl.ds(0, V - 128 * NBLK)],
            out_hbm.at[pl.ds(128 * NBLK, V - 128 * NBLK)],
        )


@functools.partial(
    pl.kernel,
    mesh=_mesh,
    out_type=jax.ShapeDtypeStruct((B // 2, 2 * D), jnp.float32),
    scratch_types=[
        pltpu.VMEM((BPW, L), jnp.int32),
        pltpu.VMEM((L, D), jnp.float32),
        pltpu.VMEM((L, D), jnp.float32),
        pltpu.VMEM((BPW // 2, 2 * D), jnp.float32),
        pltpu.SemaphoreType.DMA,
        pltpu.SemaphoreType.DMA,
    ],
)
def _emb_pool(x_hbm, table_hbm, out_hbm, idx_v, rows_a, rows_b, out_v,
              sem_a, sem_b):
    wid = lax.axis_index("s") * NC + lax.axis_index("c")
    pltpu.sync_copy(x_hbm.at[pl.ds(wid * BPW, BPW)], idx_v)

    def row_copy(q, slot, buf, sem):
        pltpu.make_async_copy(
            table_hbm.at[pl.ds(q, 1)],
            buf.at[pl.ds(slot, 1)],
            sem,
        ).start()

    def issue(elt, buf, sem):
        def issue_group(g, carry):
            base = 16 * g
            q16 = idx_v[elt, pl.ds(base, 16)]
            for k in range(16):
                row_copy(q16[k], base + k, buf, sem)
            return carry

        lax.fori_loop(0, NG, issue_group, 0)
        q16 = idx_v[elt, pl.ds(L - 16, 16)]
        for k in range(16 - TAIL, 16):
            row_copy(q16[k], L - 16 + k, buf, sem)

    def drain(buf, sem):
        pltpu.make_async_copy(table_hbm.at[pl.ds(0, L)], buf, sem).wait()

    def accumulate(buf, row, half):
        def acc_body(j, accs):
            r = j * UN
            new = list(accs)
            for k in range(UN):
                for c in range(NCH):
                    new[c] = new[c] + buf[r + k, pl.ds(c * 16, 16)]
            return tuple(new)

        accs = lax.fori_loop(
            0, L // UN, acc_body,
            tuple(jnp.zeros((16,), jnp.float32) for _ in range(NCH)),
        )
        for c in range(NCH):
            out_v[row, pl.ds(half * D + c * 16, 16)] = accs[c] * (1.0 / L)

    issue(0, rows_a, sem_a)

    def outer(i, carry):
        b0 = 2 * i
        issue(b0 + 1, rows_b, sem_b)
        drain(rows_a, sem_a)
        accumulate(rows_a, i, 0)
        issue(jnp.minimum(b0 + 2, BPW - 1), rows_a, sem_a)
        drain(rows_b, sem_b)
        accumulate(rows_b, i, 1)
        return carry

    lax.fori_loop(0, BPW // 2, outer, 0)
    drain(rows_a, sem_a)
    pltpu.sync_copy(out_v, out_hbm.at[pl.ds(wid * (BPW // 2), BPW // 2)])


def kernel(x, table):
    tail = lax.slice(table, (128 * NBLK, 0), (V, D))
    t2 = _repack(table.T, tail)
    return _emb_pool(x.astype(jnp.int32), t2).reshape(B, D)


# speedup vs baseline: 3.2365x; 3.2365x over previous
import functools

import jax
import jax.numpy as jnp
from jax import lax
from jax.experimental import pallas as pl
from jax.experimental.pallas import tpu as pltpu
from jax.experimental.pallas import tpu_sc as plsc

B = 4096
L = 200
D = 64
NC = 2
NS = 16
NW = NC * NS
BPW = B // NW
NCH = D // 16
NG = L // 16
TAIL = L - 16 * NG
UN = 4

_mesh = plsc.VectorSubcoreMesh(core_axis_name="c", subcore_axis_name="s")


@functools.partial(
    pl.kernel,
    mesh=_mesh,
    out_type=jax.ShapeDtypeStruct((B // 2, 2 * D), jnp.float32),
    scratch_types=[
        pltpu.VMEM((BPW, L), jnp.int32),
        pltpu.VMEM((L, D), jnp.float32),
        pltpu.VMEM((L, D), jnp.float32),
        pltpu.VMEM((BPW // 2, 2 * D), jnp.float32),
        pltpu.SemaphoreType.DMA,
        pltpu.SemaphoreType.DMA,
    ],
)
def _emb_pool(x_hbm, table_hbm, out_hbm, idx_v, rows_a, rows_b, out_v,
              sem_a, sem_b):
    wid = lax.axis_index("s") * NC + lax.axis_index("c")
    pltpu.sync_copy(x_hbm.at[pl.ds(wid * BPW, BPW)], idx_v)

    def issue(elt, buf, sem):
        def issue_group(g, carry):
            base = 16 * g
            q16 = idx_v[elt, pl.ds(base, 16)]
            for k in range(16):
                pltpu.make_async_copy(
                    table_hbm.at[pl.ds(q16[k], 1)],
                    buf.at[pl.ds(base + k, 1)],
                    sem,
                ).start()
            return carry

        lax.fori_loop(0, NG, issue_group, 0)
        q16 = idx_v[elt, pl.ds(L - 16, 16)]
        for k in range(16 - TAIL, 16):
            pltpu.make_async_copy(
                table_hbm.at[pl.ds(q16[k], 1)],
                buf.at[pl.ds(L - 16 + k, 1)],
                sem,
            ).start()

    def drain(buf, sem):
        pltpu.make_async_copy(table_hbm.at[pl.ds(0, L)], buf, sem).wait()

    def accumulate(buf, row, half):
        def acc_body(j, accs):
            r = j * UN
            new = list(accs)
            for k in range(UN):
                for c in range(NCH):
                    new[c] = new[c] + buf[r + k, pl.ds(c * 16, 16)]
            return tuple(new)

        accs = lax.fori_loop(
            0, L // UN, acc_body,
            tuple(jnp.zeros((16,), jnp.float32) for _ in range(NCH)),
        )
        for c in range(NCH):
            out_v[row, pl.ds(half * D + c * 16, 16)] = accs[c] * (1.0 / L)

    issue(0, rows_a, sem_a)

    def outer(i, carry):
        b0 = 2 * i
        issue(b0 + 1, rows_b, sem_b)
        drain(rows_a, sem_a)
        accumulate(rows_a, i, 0)
        issue(jnp.minimum(b0 + 2, BPW - 1), rows_a, sem_a)
        drain(rows_b, sem_b)
        accumulate(rows_b, i, 1)
        return carry

    lax.fori_loop(0, BPW // 2, outer, 0)
    drain(rows_a, sem_a)
    pltpu.sync_copy(out_v, out_hbm.at[pl.ds(wid * (BPW // 2), BPW // 2)])


def kernel(x, table):
    return _emb_pool(x.astype(jnp.int32), table).reshape(B, D)
